# Initial kernel scaffold; baseline (speedup 1.0000x reference)
import jax
import jax.numpy as jnp
from jax.experimental import pallas as pl


def kernel(occasion_ids, season_ids, occ_table, season_table, W1, b1, W2, b2):
    raise NotImplementedError("write your pallas kernel here")


# speedup vs baseline: 2.5988x; 2.5988x over previous
import functools

import jax
import jax.numpy as jnp
from jax import lax
from jax.experimental import pallas as pl
from jax.experimental.pallas import tpu as pltpu
from jax.experimental.pallas import tpu_sc as plsc

NUM_OCCASIONS = 25
NUM_SEASONS = 4
EMB_DIM = 256
SEASON_DIM = EMB_DIM // 4
HIDDEN = 512
BATCH = 16384
NUM_COMBO = NUM_OCCASIONS * NUM_SEASONS

_NC = 2
_NS = 16
_L = 16
_NW = _NC * _NS
_BPW = BATCH // _NW
_CH = 128
_NCH = _BPW // _CH
_CPB = _CH // _L


def _table_body(occ_ref, sea_ref, w1_ref, b1_ref, w2_ref, b2_ref, o_ref):
    hi = jax.lax.Precision.HIGHEST
    a = lax.dot_general(
        occ_ref[...], w1_ref[:, :EMB_DIM],
        (((1,), (1,)), ((), ())),
        preferred_element_type=jnp.float32, precision=hi)
    b = lax.dot_general(
        sea_ref[...], w1_ref[:, EMB_DIM:],
        (((1,), (1,)), ((), ())),
        preferred_element_type=jnp.float32, precision=hi)
    rows_o = lax.broadcasted_iota(jnp.int32, (NUM_COMBO, NUM_OCCASIONS), 0)
    cols_o = lax.broadcasted_iota(jnp.int32, (NUM_COMBO, NUM_OCCASIONS), 1)
    r = (rows_o // NUM_SEASONS == cols_o).astype(jnp.float32)
    rows_s = lax.broadcasted_iota(jnp.int32, (NUM_COMBO, NUM_SEASONS), 0)
    cols_s = lax.broadcasted_iota(jnp.int32, (NUM_COMBO, NUM_SEASONS), 1)
    s = (rows_s % NUM_SEASONS == cols_s).astype(jnp.float32)
    h = (
        lax.dot_general(r, a, (((1,), (0,)), ((), ())),
                        preferred_element_type=jnp.float32, precision=hi)
        + lax.dot_general(s, b, (((1,), (0,)), ((), ())),
                          preferred_element_type=jnp.float32, precision=hi)
        + b1_ref[...]
    )
    g = 0.5 * h * (1.0 + lax.erf(h * (2.0 ** -0.5)))
    o_ref[...] = (
        lax.dot_general(g, w2_ref[...], (((1,), (1,)), ((), ())),
                        preferred_element_type=jnp.float32, precision=hi)
        + b2_ref[...]
    )


def _build_table(occ_table, season_table, W1, b1r, W2, b2r):
    return pl.pallas_call(
        _table_body,
        out_shape=jax.ShapeDtypeStruct((NUM_COMBO, HIDDEN), jnp.float32),
    )(occ_table, season_table, W1, b1r, W2, b2r)


def _gather_body(tab_hbm, occ_hbm, sea_hbm, out_hbm, occ_v, sea_v, idx_v,
                 rows_v, sem):
    wid = lax.axis_index("s") * _NC + lax.axis_index("c")
    base = wid * _BPW
    pltpu.sync_copy(occ_hbm.at[pl.ds(base, _BPW)], occ_v)
    pltpu.sync_copy(sea_hbm.at[pl.ds(base, _BPW)], sea_v)
    for j in range(_BPW // _L):
        o = occ_v[pl.ds(j * _L, _L)]
        t = sea_v[pl.ds(j * _L, _L)]
        o = jnp.clip(o, 0, NUM_OCCASIONS - 1)
        t = jnp.clip(t, 0, NUM_SEASONS - 1)
        idx_v[j // _CPB, pl.ds((j % _CPB) * _L, _L)] = o * NUM_SEASONS + t
    for c in range(_NCH):
        pltpu.async_copy(tab_hbm.at[idx_v.at[c]], rows_v, sem).wait()
        pltpu.sync_copy(rows_v, out_hbm.at[pl.ds(base + c * _CH, _CH)])


@functools.partial(jax.jit, static_argnums=())
def _gather(table, occ_ids, sea_ids):
    mesh = plsc.VectorSubcoreMesh(core_axis_name="c", subcore_axis_name="s",
                                  num_cores=_NC, num_subcores=_NS)
    return pl.kernel(
        _gather_body,
        out_type=jax.ShapeDtypeStruct((BATCH, HIDDEN), jnp.float32),
        mesh=mesh,
        scratch_types=[
            pltpu.VMEM((_BPW,), jnp.int32),
            pltpu.VMEM((_BPW,), jnp.int32),
            pltpu.VMEM((_NCH, _CH), jnp.int32),
            pltpu.VMEM((_CH, HIDDEN), jnp.float32),
            pltpu.SemaphoreType.DMA,
        ],
    )(table, occ_ids, sea_ids)


def kernel(occasion_ids, season_ids, occ_table, season_table, W1, b1, W2, b2):
    table = _build_table(
        occ_table, season_table, W1,
        b1.reshape(1, HIDDEN), W2, b2.reshape(1, HIDDEN))
    occ = occasion_ids.astype(jnp.int32)
    sea = season_ids.astype(jnp.int32)
    return _gather(table, occ, sea)


# speedup vs baseline: 2.7892x; 1.0733x over previous
import jax
import jax.numpy as jnp
from jax import lax
from jax.experimental import pallas as pl
from jax.experimental.pallas import tpu as pltpu
from jax.experimental.pallas import tpu_sc as plsc

NUM_OCCASIONS = 25
NUM_SEASONS = 4
EMB_DIM = 256
SEASON_DIM = EMB_DIM // 4
HIDDEN = 512
BATCH = 16384
NUM_COMBO = NUM_OCCASIONS * NUM_SEASONS

_NC = 2
_NS = 16
_NW = _NC * _NS
_BPW = BATCH // _NW
_CH = 64
_NCH = _BPW // _CH


def _table_body(occ_ids_ref, sea_ids_ref, occ_ref, sea_ref, w1_ref, b1_ref,
                w2_ref, b2_ref, o_ref, idx_ref):
    oi = jnp.clip(occ_ids_ref[...], 0, NUM_OCCASIONS - 1)
    si = jnp.clip(sea_ids_ref[...], 0, NUM_SEASONS - 1)
    idx_ref[...] = oi * NUM_SEASONS + si
    hi = jax.lax.Precision.HIGHEST
    a = lax.dot_general(
        occ_ref[...], w1_ref[:, :EMB_DIM],
        (((1,), (1,)), ((), ())),
        preferred_element_type=jnp.float32, precision=hi)
    b = lax.dot_general(
        sea_ref[...], w1_ref[:, EMB_DIM:],
        (((1,), (1,)), ((), ())),
        preferred_element_type=jnp.float32, precision=hi)
    rows_o = lax.broadcasted_iota(jnp.int32, (NUM_COMBO, NUM_OCCASIONS), 0)
    cols_o = lax.broadcasted_iota(jnp.int32, (NUM_COMBO, NUM_OCCASIONS), 1)
    r = (rows_o // NUM_SEASONS == cols_o).astype(jnp.float32)
    rows_s = lax.broadcasted_iota(jnp.int32, (NUM_COMBO, NUM_SEASONS), 0)
    cols_s = lax.broadcasted_iota(jnp.int32, (NUM_COMBO, NUM_SEASONS), 1)
    s = (rows_s % NUM_SEASONS == cols_s).astype(jnp.float32)
    h = (
        lax.dot_general(r, a, (((1,), (0,)), ((), ())),
                        preferred_element_type=jnp.float32, precision=hi)
        + lax.dot_general(s, b, (((1,), (0,)), ((), ())),
                          preferred_element_type=jnp.float32, precision=hi)
        + b1_ref[...]
    )
    g = 0.5 * h * (1.0 + lax.erf(h * (2.0 ** -0.5)))
    o_ref[...] = (
        lax.dot_general(g, w2_ref[...], (((1,), (1,)), ((), ())),
                        preferred_element_type=jnp.float32, precision=hi)
        + b2_ref[...]
    )


def _build_table(occ_ids2d, sea_ids2d, occ_table, season_table, W1, b1r, W2,
                 b2r):
    return pl.pallas_call(
        _table_body,
        out_shape=(
            jax.ShapeDtypeStruct((NUM_COMBO, HIDDEN), jnp.float32),
            jax.ShapeDtypeStruct(occ_ids2d.shape, jnp.int32),
        ),
    )(occ_ids2d, sea_ids2d, occ_table, season_table, W1, b1r, W2, b2r)


def _gather_body(tab_hbm, idx_hbm, out_hbm, idx_v, buf0, buf1, sg0, sg1, so0,
                 so1):
    wid = lax.axis_index("s") * _NC + lax.axis_index("c")
    base = wid * _BPW
    pltpu.sync_copy(idx_hbm.at[pl.ds(base, _BPW)], idx_v)
    bufs = (buf0, buf1)
    gsems = (sg0, sg1)
    osems = (so0, so1)
    gcp = [None, None]
    ocp = [None, None]
    gcp[0] = pltpu.async_copy(
        tab_hbm.at[idx_v.at[pl.ds(0, _CH)]], buf0, sg0)
    for c in range(_NCH):
        b = c & 1
        gcp[b].wait()
        if c + 1 < _NCH:
            nb = b ^ 1
            if c >= 1:
                ocp[nb].wait()
            gcp[nb] = pltpu.async_copy(
                tab_hbm.at[idx_v.at[pl.ds((c + 1) * _CH, _CH)]],
                bufs[nb], gsems[nb])
        ocp[b] = pltpu.async_copy(
            bufs[b], out_hbm.at[pl.ds(base + c * _CH, _CH)], osems[b])
    ocp[(_NCH - 1) & 1].wait()
    ocp[(_NCH - 2) & 1].wait()


def _gather(table, idx):
    mesh = plsc.VectorSubcoreMesh(core_axis_name="c", subcore_axis_name="s",
                                  num_cores=_NC, num_subcores=_NS)
    return pl.kernel(
        _gather_body,
        out_type=jax.ShapeDtypeStruct((BATCH, HIDDEN), jnp.float32),
        mesh=mesh,
        scratch_types=[
            pltpu.VMEM((_BPW,), jnp.int32),
            pltpu.VMEM((_CH, HIDDEN), jnp.float32),
            pltpu.VMEM((_CH, HIDDEN), jnp.float32),
            pltpu.SemaphoreType.DMA,
            pltpu.SemaphoreType.DMA,
            pltpu.SemaphoreType.DMA,
            pltpu.SemaphoreType.DMA,
        ],
    )(table, idx)


def kernel(occasion_ids, season_ids, occ_table, season_table, W1, b1, W2, b2):
    occ2d = occasion_ids.astype(jnp.int32).reshape(BATCH // 128, 128)
    sea2d = season_ids.astype(jnp.int32).reshape(BATCH // 128, 128)
    table, idx2d = _build_table(
        occ2d, sea2d, occ_table, season_table, W1,
        b1.reshape(1, HIDDEN), W2, b2.reshape(1, HIDDEN))
    return _gather(table, idx2d.reshape(BATCH))


# speedup vs baseline: 3.5383x; 1.2686x over previous
import jax
import jax.numpy as jnp
from jax import lax
from jax.experimental import pallas as pl
from jax.experimental.pallas import tpu as pltpu
from jax.experimental.pallas import tpu_sc as plsc

NUM_OCCASIONS = 25
NUM_SEASONS = 4
EMB_DIM = 256
SEASON_DIM = EMB_DIM // 4
HIDDEN = 512
BATCH = 16384
NUM_COMBO = NUM_OCCASIONS * NUM_SEASONS
TAB_ROWS = 128

_NC = 2
_NS = 16
_NW = _NC * _NS
_S = 4096
_BPW = _S // _NW
_CH = 64
_NCH = _BPW // _CH
_TCBLK = 1024
_SBLK = _S // _TCBLK


def _table_body(occ_ids_ref, sea_ids_ref, occ_ref, sea_ref, w1_ref, b1_ref,
                w2_ref, b2_ref, o_ref, idx_ref):
    oi = jnp.clip(occ_ids_ref[...], 0, NUM_OCCASIONS - 1)
    si = jnp.clip(sea_ids_ref[...], 0, NUM_SEASONS - 1)
    idx_ref[...] = oi * NUM_SEASONS + si
    hi = jax.lax.Precision.HIGHEST
    a = lax.dot_general(
        occ_ref[...], w1_ref[:, :EMB_DIM],
        (((1,), (1,)), ((), ())),
        preferred_element_type=jnp.float32, precision=hi)
    b = lax.dot_general(
        sea_ref[...], w1_ref[:, EMB_DIM:],
        (((1,), (1,)), ((), ())),
        preferred_element_type=jnp.float32, precision=hi)
    rows_o = lax.broadcasted_iota(jnp.int32, (TAB_ROWS, NUM_OCCASIONS), 0)
    cols_o = lax.broadcasted_iota(jnp.int32, (TAB_ROWS, NUM_OCCASIONS), 1)
    r = (rows_o // NUM_SEASONS == cols_o).astype(jnp.float32)
    rows_s = lax.broadcasted_iota(jnp.int32, (TAB_ROWS, NUM_SEASONS), 0)
    cols_s = lax.broadcasted_iota(jnp.int32, (TAB_ROWS, NUM_SEASONS), 1)
    s = ((rows_s % NUM_SEASONS == cols_s) & (rows_s < NUM_COMBO)
         ).astype(jnp.float32)
    h = (
        lax.dot_general(r, a, (((1,), (0,)), ((), ())),
                        preferred_element_type=jnp.float32, precision=hi)
        + lax.dot_general(s, b, (((1,), (0,)), ((), ())),
                          preferred_element_type=jnp.float32, precision=hi)
        + b1_ref[...]
    )
    g = 0.5 * h * (1.0 + lax.erf(h * (2.0 ** -0.5)))
    o_ref[...] = (
        lax.dot_general(g, w2_ref[...], (((1,), (1,)), ((), ())),
                        preferred_element_type=jnp.float32, precision=hi)
        + b2_ref[...]
    )


def _build_table(occ_ids2d, sea_ids2d, occ_table, season_table, W1, b1r, W2,
                 b2r):
    return pl.pallas_call(
        _table_body,
        out_shape=(
            jax.ShapeDtypeStruct((TAB_ROWS, HIDDEN), jnp.float32),
            jax.ShapeDtypeStruct(occ_ids2d.shape, jnp.int32),
        ),
    )(occ_ids2d, sea_ids2d, occ_table, season_table, W1, b1r, W2, b2r)


def _gather_body(tab_hbm, idx_hbm, out_hbm, idx_v, buf0, buf1, sg0, sg1, so0,
                 so1):
    wid = lax.axis_index("s") * _NC + lax.axis_index("c")
    base = wid * _BPW
    pltpu.sync_copy(idx_hbm.at[pl.ds(base, _BPW)], idx_v)
    bufs = (buf0, buf1)
    gsems = (sg0, sg1)
    osems = (so0, so1)
    gcp = [None, None]
    ocp = [None, None]
    gcp[0] = pltpu.async_copy(
        tab_hbm.at[idx_v.at[pl.ds(0, _CH)]], buf0, sg0)
    for c in range(_NCH):
        b = c & 1
        gcp[b].wait()
        if c + 1 < _NCH:
            nb = b ^ 1
            if c >= 1:
                ocp[nb].wait()
            gcp[nb] = pltpu.async_copy(
                tab_hbm.at[idx_v.at[pl.ds((c + 1) * _CH, _CH)]],
                bufs[nb], gsems[nb])
        ocp[b] = pltpu.async_copy(
            bufs[b], out_hbm.at[pl.ds(base + c * _CH, _CH)], osems[b])
    ocp[(_NCH - 1) & 1].wait()
    if _NCH > 1:
        ocp[(_NCH - 2) & 1].wait()


def _sc_gather(table, idx):
    mesh = plsc.VectorSubcoreMesh(core_axis_name="c", subcore_axis_name="s",
                                  num_cores=_NC, num_subcores=_NS)
    return pl.kernel(
        _gather_body,
        out_type=jax.ShapeDtypeStruct((BATCH, HIDDEN), jnp.float32),
        mesh=mesh,
        scratch_types=[
            pltpu.VMEM((_BPW,), jnp.int32),
            pltpu.VMEM((_CH, HIDDEN), jnp.float32),
            pltpu.VMEM((_CH, HIDDEN), jnp.float32),
            pltpu.SemaphoreType.DMA,
            pltpu.SemaphoreType.DMA,
            pltpu.SemaphoreType.DMA,
            pltpu.SemaphoreType.DMA,
        ],
    )(table, idx)


def _onehot_body(tab_ref, idx_ref, _outin_ref, out_ref):
    tab = tab_ref[...]
    combo = lax.broadcasted_iota(jnp.int32, (TAB_ROWS, 128), 0)
    for j in range(_TCBLK // 128):
        idx_row = idx_ref[j:j + 1, :]
        oh = (combo == idx_row).astype(jnp.float32)
        blk = lax.dot_general(
            oh, tab, (((0,), (0,)), ((), ())),
            preferred_element_type=jnp.float32,
            precision=jax.lax.Precision.HIGHEST)
        out_ref[j * 128:(j + 1) * 128, :] = blk


def _tc_expand(table, idx2d, out_partial):
    nblk = (BATCH - _S) // _TCBLK
    return pl.pallas_call(
        _onehot_body,
        grid=(nblk,),
        in_specs=[
            pl.BlockSpec((TAB_ROWS, HIDDEN), lambda g: (0, 0)),
            pl.BlockSpec((_TCBLK // 128, 128), lambda g: (g + _SBLK, 0)),
            pl.BlockSpec(memory_space=pl.ANY),
        ],
        out_specs=pl.BlockSpec((_TCBLK, HIDDEN), lambda g: (g + _SBLK, 0)),
        out_shape=jax.ShapeDtypeStruct((BATCH, HIDDEN), jnp.float32),
        input_output_aliases={2: 0},
    )(table, idx2d, out_partial)


def kernel(occasion_ids, season_ids, occ_table, season_table, W1, b1, W2, b2):
    occ2d = occasion_ids.astype(jnp.int32).reshape(BATCH // 128, 128)
    sea2d = season_ids.astype(jnp.int32).reshape(BATCH // 128, 128)
    table, idx2d = _build_table(
        occ2d, sea2d, occ_table, season_table, W1,
        b1.reshape(1, HIDDEN), W2, b2.reshape(1, HIDDEN))
    out_partial = _sc_gather(table, idx2d.reshape(BATCH))
    return _tc_expand(table, idx2d, out_partial)
